# baseline (speedup 1.0000x reference)
import functools
import math

import jax
import jax.numpy as jnp
from jax import lax
from jax.experimental import pallas as pl
from jax.experimental.pallas import tpu as pltpu
from jax.experimental.pallas import tpu_sc as plsc

R_CUT = 5.0
N_MAX_L = [8, 7, 6, 5, 4]
OFFS = [0, 8, 15, 21, 26]
N_SPECIES = 4
N_CENTERS = 50000
NSEG = N_CENTERS * N_SPECIES
N_EDGES = 1600000

NC = 2
NS = 16
LANES = 16

CHUNK = 80
BLOCK = 800
KPB = BLOCK // CHUNK
NB = 125
TW = 8
CPT = N_CENTERS // NS
ZROWS = 250
NRING = 2

_COLMAP = [(l, n) for l in range(5) for n in range(1, N_MAX_L[l] + 1)]
_GROUPS = [[_COLMAP[8 * g + t] if 8 * g + t < 30 else None for t in range(8)]
           for g in range(4)]

_A = math.pi / 2.0
_SIN_C = [(-1.0) ** k * _A ** (2 * k + 1) / math.factorial(2 * k + 1)
          for k in range(6)]
_COS_C = [(-1.0) ** k * _A ** (2 * k) / math.factorial(2 * k)
          for k in range(7)]


def _poly(u, coeffs):
    acc = jnp.full((LANES,), coeffs[-1], jnp.float32)
    for c in reversed(coeffs[:-1]):
        acc = acc * u + c
    return acc


def _factors(ex, ey, ez):
    r2 = ex * ex + ey * ey + ez * ez
    r2 = jnp.maximum(r2, 1e-24)
    i = plsc.bitcast(r2, jnp.int32)
    i = jnp.full((LANES,), 0x5F3759DF, jnp.int32) - lax.shift_right_logical(
        i, jnp.full((LANES,), 1, jnp.int32))
    y = plsc.bitcast(i, jnp.float32)
    for _ in range(3):
        y = y * (1.5 - 0.5 * r2 * y * y)
    r = r2 * y
    q = 1.0 / (r + 1e-12)
    x = jnp.minimum(r * (1.0 / R_CUT), 1.0)
    u = x * x
    sh = x * _poly(u, _SIN_C)
    ch = _poly(u, _COS_C)
    s1 = 2.0 * sh * ch
    c1 = 1.0 - 2.0 * sh * sh
    two_c1 = c1 + c1
    raw = [s1, two_c1 * s1]
    for _ in range(3, 9):
        raw.append(two_c1 * raw[-1] - raw[-2])
    cp = [ch]
    for _ in range(4):
        cp.append(cp[-1] * ch)
    qcp = [q * v for v in cp]
    return raw, qcp


@functools.lru_cache(maxsize=1)
def _make_sc_call():
    mesh = plsc.VectorSubcoreMesh(core_axis_name="c", subcore_axis_name="s",
                                  num_cores=NC, num_subcores=NS)
    cparams = pltpu.CompilerParams(needs_layout_passes=False,
                                   use_tc_tiling_on_sc=False)

    @pl.kernel(
        out_type=jax.ShapeDtypeStruct((4, NSEG, TW), jnp.float32),
        mesh=mesh,
        scratch_types=[
            pltpu.VMEM((2, BLOCK, 3), jnp.float32),
            pltpu.VMEM((2, BLOCK), jnp.int32),
            pltpu.VMEM((2, BLOCK), jnp.int32),
            pltpu.VMEM((KPB, CHUNK), jnp.int32),
            pltpu.VMEM((NRING, CHUNK, TW), jnp.float32),
            pltpu.VMEM((ZROWS, TW), jnp.float32),
            pltpu.VMEM_SHARED((NSEG, TW), jnp.float32),
            pltpu.SemaphoreType.DMA((2,)),
            pltpu.SemaphoreType.DMA((NRING,)),
        ],
        compiler_params=cparams,
    )
    def sc_call(ev_hbm, ci_hbm, si_hbm, out_hbm,
                ev_v, ci_v, si_v, didx_v, rows_v, zbuf_v,
                table_sh, in_sem, sc_sem):
        c = lax.axis_index("c")
        w = lax.axis_index("s")
        cvec = jnp.full((LANES,), 1.0, jnp.float32) * lax.convert_element_type(
            c, jnp.float32)
        lane = lax.iota(jnp.int32, LANES)
        zeros16 = jnp.zeros((LANES,), jnp.float32)
        comp0 = jnp.zeros((LANES,), jnp.int32)
        comp1 = jnp.full((LANES,), 1, jnp.int32)
        comp2 = jnp.full((LANES,), 2, jnp.int32)
        tfull = [jnp.full((LANES,), t, jnp.int32) for t in range(TW)]

        def in_copies(b, d):
            base = (w * NB + b) * BLOCK
            sl = pl.ds(base, BLOCK)
            return [
                pltpu.make_async_copy(ev_hbm.at[sl], ev_v.at[d], in_sem.at[d]),
                pltpu.make_async_copy(ci_hbm.at[sl], ci_v.at[d], in_sem.at[d]),
                pltpu.make_async_copy(si_hbm.at[sl], si_v.at[d], in_sem.at[d]),
            ]

        rhalf = lax.shift_right_logical(lane, jnp.full((LANES,), 3, jnp.int32))
        c8 = lane & 7

        @pl.loop(0, ZROWS // 2)
        def _(i):
            plsc.store_scatter(zbuf_v, [i * 2 + rhalf, c8], zeros16)

        for p in range(2):
            nz = (NSEG // NS) // ZROWS

            @pl.loop(0, nz)
            def _(i):
                pltpu.sync_copy(
                    zbuf_v,
                    table_sh.at[pl.ds((w * nz + i) * ZROWS, ZROWS)])

            plsc.subcore_barrier()

            for d in range(2):
                for cp_ in in_copies(d, d):
                    cp_.start()

            ga, gb = _GROUPS[p], _GROUPS[2 + p]

            def do_block(b, d):
                    for cp_ in in_copies(b, d):
                        cp_.wait()

                    def do_chunk(k, qq, wait_pred):
                        def _w():
                            pltpu.make_async_copy(
                                rows_v.at[qq],
                                table_sh.at[didx_v.at[k - NRING]],
                                sc_sem.at[qq]).wait()

                        if wait_pred is True:
                            _w()
                        elif wait_pred is not None:
                            pl.when(wait_pred)(_w)

                        for j in range(CHUNK // LANES):
                            o = k * CHUNK + j * LANES
                            cidx = ci_v[d, pl.ds(o, LANES)]
                            sidx = si_v[d, pl.ds(o, LANES)]
                            didx_v[k, pl.ds(j * LANES, LANES)] = (
                                cidx * N_SPECIES + sidx)
                            rows = o + lane
                            ex = plsc.load_gather(ev_v.at[d], [rows, comp0])
                            ey = plsc.load_gather(ev_v.at[d], [rows, comp1])
                            ez = plsc.load_gather(ev_v.at[d], [rows, comp2])
                            raw, qcp = _factors(ex, ey, ez)
                            ridx = j * LANES + lane
                            for t in range(TW):
                                la, na = ga[t]
                                m1 = raw[na - 1] * qcp[la]
                                if gb[t] is None:
                                    val = m1
                                else:
                                    lb, nb = gb[t]
                                    m2 = raw[nb - 1] * qcp[lb]
                                    val = m1 + cvec * (m2 - m1)
                                plsc.store_scatter(
                                    rows_v.at[qq], [ridx, tfull[t]], val)
                        pltpu.async_copy(
                            rows_v.at[qq], table_sh.at[didx_v.at[k]],
                            sc_sem.at[qq], add=True)

                    @pl.loop(0, KPB, step=NRING)
                    def _(k0):
                        do_chunk(k0, 0, k0 > 0)
                        do_chunk(k0 + 1, 1, k0 > 0)

                    pltpu.make_async_copy(
                        rows_v.at[0], table_sh.at[didx_v.at[KPB - 2]],
                        sc_sem.at[0]).wait()
                    pltpu.make_async_copy(
                        rows_v.at[1], table_sh.at[didx_v.at[KPB - 1]],
                        sc_sem.at[1]).wait()

                    @pl.when(b + 2 < NB)
                    def _():
                        for cp_ in in_copies(b + 2, d):
                            cp_.start()

            @pl.loop(0, NB - 1, step=2)
            def _(b0):
                do_block(b0, 0)
                do_block(b0 + 1, 1)

            do_block(NB - 1, 0)

            plsc.subcore_barrier()

            g = 2 * c + p
            rpt = NSEG // NS
            pltpu.sync_copy(
                table_sh.at[pl.ds(w * rpt, rpt)],
                out_hbm.at[g, pl.ds(w * rpt, rpt)])

            plsc.subcore_barrier()

    return sc_call


def kernel(edge_vec, center_index, neighbor_species_index):
    tbl = _make_sc_call()(edge_vec, center_index, neighbor_species_index)
    d = tbl.transpose(1, 0, 2).reshape(NSEG, 32)[:, :30]
    d = d.reshape(N_CENTERS, N_SPECIES, 30)
    return jnp.concatenate(
        [d[:, :, OFFS[l]:OFFS[l] + N_MAX_L[l]].reshape(
            N_CENTERS, N_SPECIES * N_MAX_L[l]) for l in range(5)],
        axis=1)


# speedup vs baseline: 1.0651x; 1.0651x over previous
import functools
import math

import jax
import jax.numpy as jnp
from jax import lax
from jax.experimental import pallas as pl
from jax.experimental.pallas import tpu as pltpu
from jax.experimental.pallas import tpu_sc as plsc

R_CUT = 5.0
N_MAX_L = [8, 7, 6, 5, 4]
OFFS = [0, 8, 15, 21, 26]
N_SPECIES = 4
N_CENTERS = 50000
NSEG = N_CENTERS * N_SPECIES
N_EDGES = 1600000

NC = 2
NS = 16
LANES = 16

CHUNK = 80
BLOCK = 800
KPB = BLOCK // CHUNK
NB = 125
TW = 8
CPT = N_CENTERS // NS
ZROWS = 250
NRING = 2

_COLMAP = [(l, n) for l in range(5) for n in range(1, N_MAX_L[l] + 1)]
_GROUPS = [[_COLMAP[8 * g + t] if 8 * g + t < 30 else None for t in range(8)]
           for g in range(4)]

_A = math.pi / 2.0
_SIN_C = [(-1.0) ** k * _A ** (2 * k + 1) / math.factorial(2 * k + 1)
          for k in range(6)]
_COS_C = [(-1.0) ** k * _A ** (2 * k) / math.factorial(2 * k)
          for k in range(7)]


def _poly(u, coeffs):
    acc = jnp.full((LANES,), coeffs[-1], jnp.float32)
    for c in reversed(coeffs[:-1]):
        acc = acc * u + c
    return acc


def _factors(ex, ey, ez):
    r2 = ex * ex + ey * ey + ez * ez
    r2 = jnp.maximum(r2, 1e-24)
    i = plsc.bitcast(r2, jnp.int32)
    i = jnp.full((LANES,), 0x5F3759DF, jnp.int32) - lax.shift_right_logical(
        i, jnp.full((LANES,), 1, jnp.int32))
    y = plsc.bitcast(i, jnp.float32)
    for _ in range(3):
        y = y * (1.5 - 0.5 * r2 * y * y)
    r = r2 * y
    q = 1.0 / (r + 1e-12)
    x = jnp.minimum(r * (1.0 / R_CUT), 1.0)
    u = x * x
    sh = x * _poly(u, _SIN_C)
    ch = _poly(u, _COS_C)
    s1 = 2.0 * sh * ch
    c1 = 1.0 - 2.0 * sh * sh
    two_c1 = c1 + c1
    raw = [s1, two_c1 * s1]
    for _ in range(3, 9):
        raw.append(two_c1 * raw[-1] - raw[-2])
    cp = [ch]
    for _ in range(4):
        cp.append(cp[-1] * ch)
    qcp = [q * v for v in cp]
    return raw, qcp


@functools.lru_cache(maxsize=1)
def _make_sc_call():
    mesh = plsc.VectorSubcoreMesh(core_axis_name="c", subcore_axis_name="s",
                                  num_cores=NC, num_subcores=NS)
    cparams = pltpu.CompilerParams(needs_layout_passes=False,
                                   use_tc_tiling_on_sc=False)

    @pl.kernel(
        out_type=jax.ShapeDtypeStruct((4, NSEG, TW), jnp.float32),
        mesh=mesh,
        scratch_types=[
            pltpu.VMEM((2, BLOCK * 3), jnp.float32),
            pltpu.VMEM((2, BLOCK), jnp.int32),
            pltpu.VMEM((2, BLOCK), jnp.int32),
            pltpu.VMEM((KPB, CHUNK), jnp.int32),
            pltpu.VMEM((NRING, CHUNK, TW), jnp.float32),
            pltpu.VMEM((ZROWS, TW), jnp.float32),
            pltpu.VMEM_SHARED((NSEG, TW), jnp.float32),
            pltpu.SemaphoreType.DMA((2,)),
            pltpu.SemaphoreType.DMA((NRING,)),
        ],
        compiler_params=cparams,
    )
    def sc_call(ev_hbm, ci_hbm, si_hbm, out_hbm,
                ev_v, ci_v, si_v, didx_v, rows_v, zbuf_v,
                table_sh, in_sem, sc_sem):
        c = lax.axis_index("c")
        w = lax.axis_index("s")
        cvec = jnp.full((LANES,), 1.0, jnp.float32) * lax.convert_element_type(
            c, jnp.float32)
        lane = lax.iota(jnp.int32, LANES)
        zeros16 = jnp.zeros((LANES,), jnp.float32)
        lane3 = lane * 3
        tfull = [jnp.full((LANES,), t, jnp.int32) for t in range(TW)]

        def in_copies(b, d):
            base = (w * NB + b) * BLOCK
            sl = pl.ds(base, BLOCK)
            sl3 = pl.ds(base * 3, BLOCK * 3)
            return [
                pltpu.make_async_copy(ev_hbm.at[sl3], ev_v.at[d], in_sem.at[d]),
                pltpu.make_async_copy(ci_hbm.at[sl], ci_v.at[d], in_sem.at[d]),
                pltpu.make_async_copy(si_hbm.at[sl], si_v.at[d], in_sem.at[d]),
            ]

        rhalf = lax.shift_right_logical(lane, jnp.full((LANES,), 3, jnp.int32))
        c8 = lane & 7

        @pl.loop(0, ZROWS // 2)
        def _(i):
            plsc.store_scatter(zbuf_v, [i * 2 + rhalf, c8], zeros16)

        for p in range(2):
            nz = (NSEG // NS) // ZROWS

            @pl.loop(0, nz)
            def _(i):
                pltpu.sync_copy(
                    zbuf_v,
                    table_sh.at[pl.ds((w * nz + i) * ZROWS, ZROWS)])

            plsc.subcore_barrier()

            for d in range(2):
                for cp_ in in_copies(d, d):
                    cp_.start()

            ga, gb = _GROUPS[p], _GROUPS[2 + p]

            def do_block(b, d):
                    for cp_ in in_copies(b, d):
                        cp_.wait()

                    def do_chunk(k, qq, wait_pred):
                        def _w():
                            pltpu.make_async_copy(
                                rows_v.at[qq],
                                table_sh.at[didx_v.at[k - NRING]],
                                sc_sem.at[qq]).wait()

                        if wait_pred is True:
                            _w()
                        elif wait_pred is not None:
                            pl.when(wait_pred)(_w)

                        for j in range(CHUNK // LANES):
                            o = k * CHUNK + j * LANES
                            cidx = ci_v[d, pl.ds(o, LANES)]
                            sidx = si_v[d, pl.ds(o, LANES)]
                            didx_v[k, pl.ds(j * LANES, LANES)] = (
                                cidx * N_SPECIES + sidx)
                            fidx = o * 3 + lane3
                            ex = plsc.load_gather(ev_v.at[d], [fidx])
                            ey = plsc.load_gather(ev_v.at[d], [fidx + 1])
                            ez = plsc.load_gather(ev_v.at[d], [fidx + 2])
                            raw, qcp = _factors(ex, ey, ez)
                            ridx = j * LANES + lane
                            for t in range(TW):
                                la, na = ga[t]
                                m1 = raw[na - 1] * qcp[la]
                                if gb[t] is None:
                                    val = m1
                                else:
                                    lb, nb = gb[t]
                                    m2 = raw[nb - 1] * qcp[lb]
                                    val = m1 + cvec * (m2 - m1)
                                plsc.store_scatter(
                                    rows_v.at[qq], [ridx, tfull[t]], val)
                        pltpu.async_copy(
                            rows_v.at[qq], table_sh.at[didx_v.at[k]],
                            sc_sem.at[qq], add=True)

                    @pl.loop(0, KPB, step=NRING)
                    def _(k0):
                        do_chunk(k0, 0, k0 > 0)
                        do_chunk(k0 + 1, 1, k0 > 0)

                    pltpu.make_async_copy(
                        rows_v.at[0], table_sh.at[didx_v.at[KPB - 2]],
                        sc_sem.at[0]).wait()
                    pltpu.make_async_copy(
                        rows_v.at[1], table_sh.at[didx_v.at[KPB - 1]],
                        sc_sem.at[1]).wait()

                    @pl.when(b + 2 < NB)
                    def _():
                        for cp_ in in_copies(b + 2, d):
                            cp_.start()

            @pl.loop(0, NB - 1, step=2)
            def _(b0):
                do_block(b0, 0)
                do_block(b0 + 1, 1)

            do_block(NB - 1, 0)

            plsc.subcore_barrier()

            g = 2 * c + p
            rpt = NSEG // NS
            pltpu.sync_copy(
                table_sh.at[pl.ds(w * rpt, rpt)],
                out_hbm.at[g, pl.ds(w * rpt, rpt)])

            plsc.subcore_barrier()

    return sc_call


def kernel(edge_vec, center_index, neighbor_species_index):
    tbl = _make_sc_call()(edge_vec.reshape(-1), center_index,
                          neighbor_species_index)
    d = tbl.transpose(1, 0, 2).reshape(NSEG, 32)[:, :30]
    d = d.reshape(N_CENTERS, N_SPECIES, 30)
    return jnp.concatenate(
        [d[:, :, OFFS[l]:OFFS[l] + N_MAX_L[l]].reshape(
            N_CENTERS, N_SPECIES * N_MAX_L[l]) for l in range(5)],
        axis=1)


# speedup vs baseline: 7.2149x; 6.7736x over previous
import functools
import math

import jax
import jax.numpy as jnp
from jax import lax
from jax.experimental import pallas as pl
from jax.experimental.pallas import tpu as pltpu
from jax.experimental.pallas import tpu_sc as plsc

R_CUT = 5.0
N_MAX_L = [8, 7, 6, 5, 4]
OFFS = [0, 8, 15, 21, 26]
N_SPECIES = 4
N_CENTERS = 50000
NSEG = N_CENTERS * N_SPECIES
N_EDGES = 1600000

NC = 2
NS = 16
LANES = 16

CHUNK = 80
BLOCK = 4000
KPB = BLOCK // CHUNK
NB = 25
TW = 8
ZROWS = 100
NRING = 2

_COLMAP = [(l, n) for l in range(5) for n in range(1, N_MAX_L[l] + 1)]
_GROUPS = [[_COLMAP[8 * g + t] if 8 * g + t < 30 else None for t in range(8)]
           for g in range(4)]

_A = math.pi / 2.0
_SIN_C = [(-1.0) ** k * _A ** (2 * k + 1) / math.factorial(2 * k + 1)
          for k in range(6)]
_COS_C = [(-1.0) ** k * _A ** (2 * k) / math.factorial(2 * k)
          for k in range(7)]


def _poly(u, coeffs):
    acc = jnp.full((LANES,), coeffs[-1], jnp.float32)
    for c in reversed(coeffs[:-1]):
        acc = acc * u + c
    return acc


def _factors(r2):
    r2 = jnp.maximum(r2, 1e-24)
    i = plsc.bitcast(r2, jnp.int32)
    i = jnp.full((LANES,), 0x5F3759DF, jnp.int32) - lax.shift_right_logical(
        i, jnp.full((LANES,), 1, jnp.int32))
    y = plsc.bitcast(i, jnp.float32)
    for _ in range(3):
        y = y * (1.5 - 0.5 * r2 * y * y)
    r = r2 * y
    q = 1.0 / (r + 1e-12)
    x = jnp.minimum(r * (1.0 / R_CUT), 1.0)
    u = x * x
    sh = x * _poly(u, _SIN_C)
    ch = _poly(u, _COS_C)
    s1 = 2.0 * sh * ch
    c1 = 1.0 - 2.0 * sh * sh
    two_c1 = c1 + c1
    raw = [s1, two_c1 * s1]
    for _ in range(3, 9):
        raw.append(two_c1 * raw[-1] - raw[-2])
    cp = [ch]
    for _ in range(4):
        cp.append(cp[-1] * ch)
    qcp = [q * v for v in cp]
    return raw, qcp


@functools.lru_cache(maxsize=1)
def _make_sc_call():
    mesh = plsc.VectorSubcoreMesh(core_axis_name="c", subcore_axis_name="s",
                                  num_cores=NC, num_subcores=NS)
    cparams = pltpu.CompilerParams(needs_layout_passes=False,
                                   use_tc_tiling_on_sc=False)

    @pl.kernel(
        out_type=jax.ShapeDtypeStruct((NSEG, 4 * TW), jnp.float32),
        mesh=mesh,
        scratch_types=[
            pltpu.VMEM((2, BLOCK), jnp.float32),
            pltpu.VMEM((2, BLOCK), jnp.int32),
            pltpu.VMEM((2, BLOCK), jnp.int32),
            pltpu.VMEM((KPB, CHUNK), jnp.int32),
            pltpu.VMEM((NRING, CHUNK, TW), jnp.float32),
            pltpu.VMEM((ZROWS, TW), jnp.float32),
            pltpu.VMEM_SHARED((NSEG, TW), jnp.float32),
            pltpu.SemaphoreType.DMA((2,)),
            pltpu.SemaphoreType.DMA((NRING,)),
        ],
        compiler_params=cparams,
    )
    def sc_call(r2_hbm, ci_hbm, si_hbm, out_hbm,
                r2_v, ci_v, si_v, didx_v, rows_v, zbuf_v,
                table_sh, in_sem, sc_sem):
        c = lax.axis_index("c")
        w = lax.axis_index("s")
        cvec = jnp.full((LANES,), 1.0, jnp.float32) * lax.convert_element_type(
            c, jnp.float32)
        lane = lax.iota(jnp.int32, LANES)
        zeros16 = jnp.zeros((LANES,), jnp.float32)
        tfull = [jnp.full((LANES,), t, jnp.int32) for t in range(TW)]

        def in_copies(b, d):
            base = (w * NB + b) * BLOCK
            sl = pl.ds(base, BLOCK)
            return [
                pltpu.make_async_copy(r2_hbm.at[sl], r2_v.at[d], in_sem.at[d]),
                pltpu.make_async_copy(ci_hbm.at[sl], ci_v.at[d], in_sem.at[d]),
                pltpu.make_async_copy(si_hbm.at[sl], si_v.at[d], in_sem.at[d]),
            ]

        rhalf = lax.shift_right_logical(lane, jnp.full((LANES,), 3, jnp.int32))
        c8 = lane & 7

        @pl.loop(0, ZROWS // 2)
        def _(i):
            plsc.store_scatter(zbuf_v, [i * 2 + rhalf, c8], zeros16)

        for p in range(2):
            nz = (NSEG // NS) // ZROWS

            @pl.loop(0, nz)
            def _(i):
                pltpu.sync_copy(
                    zbuf_v,
                    table_sh.at[pl.ds((w * nz + i) * ZROWS, ZROWS)])

            plsc.subcore_barrier()

            for d in range(2):
                for cp_ in in_copies(d, d):
                    cp_.start()

            ga, gb = _GROUPS[p], _GROUPS[2 + p]

            def do_block(b, d):
                    for cp_ in in_copies(b, d):
                        cp_.wait()

                    def do_chunk(k, qq, wait_pred):
                        def _w():
                            pltpu.make_async_copy(
                                rows_v.at[qq],
                                table_sh.at[didx_v.at[k - NRING]],
                                sc_sem.at[qq]).wait()

                        if wait_pred is True:
                            _w()
                        elif wait_pred is not None:
                            pl.when(wait_pred)(_w)

                        for j in range(CHUNK // LANES):
                            o = k * CHUNK + j * LANES
                            cidx = ci_v[d, pl.ds(o, LANES)]
                            sidx = si_v[d, pl.ds(o, LANES)]
                            didx_v[k, pl.ds(j * LANES, LANES)] = (
                                cidx * N_SPECIES + sidx)
                            raw, qcp = _factors(r2_v[d, pl.ds(o, LANES)])
                            ridx = j * LANES + lane
                            for t in range(TW):
                                la, na = ga[t]
                                m1 = raw[na - 1] * qcp[la]
                                if gb[t] is None:
                                    val = m1
                                else:
                                    lb, nb = gb[t]
                                    m2 = raw[nb - 1] * qcp[lb]
                                    val = m1 + cvec * (m2 - m1)
                                plsc.store_scatter(
                                    rows_v.at[qq], [ridx, tfull[t]], val)
                        pltpu.async_copy(
                            rows_v.at[qq], table_sh.at[didx_v.at[k]],
                            sc_sem.at[qq], add=True)

                    @pl.loop(0, KPB, step=NRING)
                    def _(k0):
                        do_chunk(k0, 0, k0 > 0)
                        do_chunk(k0 + 1, 1, k0 > 0)

                    pltpu.make_async_copy(
                        rows_v.at[0], table_sh.at[didx_v.at[KPB - 2]],
                        sc_sem.at[0]).wait()
                    pltpu.make_async_copy(
                        rows_v.at[1], table_sh.at[didx_v.at[KPB - 1]],
                        sc_sem.at[1]).wait()

                    @pl.when(b + 2 < NB)
                    def _():
                        for cp_ in in_copies(b + 2, d):
                            cp_.start()

            @pl.loop(0, NB - 1, step=2)
            def _(b0):
                do_block(b0, 0)
                do_block(b0 + 1, 1)

            do_block(NB - 1, 0)

            plsc.subcore_barrier()

            g = 2 * c + p
            rpt = NSEG // NS
            pltpu.sync_copy(
                table_sh.at[pl.ds(w * rpt, rpt)],
                out_hbm.at[pl.ds(w * rpt, rpt), pl.ds(g * TW, TW)])

            plsc.subcore_barrier()

    return sc_call


_PERM = [s * 32 + OFFS[l] + n
         for l in range(5) for s in range(N_SPECIES) for n in range(N_MAX_L[l])]


def kernel(edge_vec, center_index, neighbor_species_index):
    r2 = jnp.sum(edge_vec * edge_vec, axis=1)
    tbl = _make_sc_call()(r2, center_index, neighbor_species_index)
    return jnp.take(tbl.reshape(N_CENTERS, 4 * 4 * TW),
                    jnp.asarray(_PERM, dtype=jnp.int32), axis=1)
